# Initial kernel scaffold; baseline (speedup 1.0000x reference)
import jax
import jax.numpy as jnp
from jax import lax
from jax.experimental import pallas as pl
from jax.experimental.pallas import tpu as pltpu
from jax.experimental.pallas import tpu_sc as plsc

N = 10000
NP = 10240
E = 320000
D_FEAT = 128
EMBED = 64
NC = 2
NS = 16
NW = NC * NS
CHUNK = 128
CPW = 80
EP = NW * CPW * CHUNK
RPS = NP // NS
BR = 1280
F32 = jnp.float32


def _make_sc_spmm(with_deg: bool):
    mesh = plsc.VectorSubcoreMesh(core_axis_name="c", subcore_axis_name="s")
    out_type = [jax.ShapeDtypeStruct((NC, NP, EMBED), F32)]
    scratch = [
        pltpu.VMEM_SHARED((NP, EMBED), F32),
        pltpu.VMEM((CHUNK,), jnp.int32),
        pltpu.VMEM((CHUNK,), jnp.int32),
        pltpu.VMEM((CHUNK, EMBED), F32),
        pltpu.SemaphoreType.DMA,
    ]
    if with_deg:
        out_type.append(jax.ShapeDtypeStruct((NC, NP, 8), F32))
        scratch += [
            pltpu.VMEM_SHARED((NP, 8), F32),
            pltpu.VMEM((CHUNK, 8), F32),
        ]

    def body(h_hbm, src_hbm, dst_hbm, z64_hbm, z8_hbm, ones_hbm, *rest):
        if with_deg:
            (pooled_out, deg_out, pooled_sh, sidx, didx, rows, sem,
             deg_sh, ones_v) = rest
        else:
            pooled_out, pooled_sh, sidx, didx, rows, sem = rest
        cid = lax.axis_index("c")
        sid = lax.axis_index("s")
        wid = sid * NC + cid
        r0 = sid * RPS

        pltpu.sync_copy(z64_hbm, pooled_sh.at[pl.ds(r0, RPS)])
        if with_deg:
            pltpu.sync_copy(z8_hbm, deg_sh.at[pl.ds(r0, RPS)])
            pltpu.sync_copy(ones_hbm, ones_v)
        plsc.subcore_barrier()

        def step(j, carry):
            pltpu.sync_copy(src_hbm.at[wid, j], sidx)
            pltpu.async_copy(h_hbm.at[sidx], rows, sem).wait()
            pltpu.sync_copy(dst_hbm.at[wid, j], didx)
            pltpu.sync_copy(rows, pooled_sh.at[didx], add=True)
            if with_deg:
                pltpu.sync_copy(ones_v, deg_sh.at[didx], add=True)
            return carry

        lax.fori_loop(0, CPW, step, 0)
        plsc.subcore_barrier()

        pltpu.sync_copy(pooled_sh.at[pl.ds(r0, RPS)],
                        pooled_out.at[cid, pl.ds(r0, RPS)])
        if with_deg:
            pltpu.sync_copy(deg_sh.at[pl.ds(r0, RPS)],
                            deg_out.at[cid, pl.ds(r0, RPS)])

    return pl.kernel(body, out_type=out_type, mesh=mesh,
                     scratch_types=scratch)


_sc_spmm_deg = _make_sc_spmm(True)
_sc_spmm = _make_sc_spmm(False)


def _embed_body(pk_ref, nf_ref, w_ref, b_ref, bp_ref, x_ref, h_ref):
    i = pl.program_id(0)
    x = jnp.dot(nf_ref[...], w_ref[...], preferred_element_type=F32) + b_ref[...]
    gid = i * BR + lax.broadcasted_iota(jnp.int32, (BR, 1), 0)
    x = x + jnp.where(gid == pk_ref[0], 1.0, 0.0) * bp_ref[...]
    x_ref[...] = x
    h_ref[...] = jnp.maximum(x, 0.0)


def _tc_embed(nf, w, b, bp, pk):
    return pl.pallas_call(
        _embed_body,
        grid=(NP // BR,),
        in_specs=[
            pl.BlockSpec(memory_space=pltpu.SMEM),
            pl.BlockSpec((BR, D_FEAT), lambda i: (i, 0)),
            pl.BlockSpec((D_FEAT, EMBED), lambda i: (0, 0)),
            pl.BlockSpec((1, EMBED), lambda i: (0, 0)),
            pl.BlockSpec((1, EMBED), lambda i: (0, 0)),
        ],
        out_specs=[pl.BlockSpec((BR, EMBED), lambda i: (i, 0))] * 2,
        out_shape=[jax.ShapeDtypeStruct((NP, EMBED), F32)] * 2,
    )(pk, nf, w, b, bp)


def _update_body(p_ref, d_ref, cw_ref, cb_ref, x_ref, h_ref):
    pooled = p_ref[0] + p_ref[1]
    deg = d_ref[0][:, 0:1] + d_ref[1][:, 0:1]
    coeff = 1.0 / jnp.maximum(deg, 1.0)
    nl = jnp.dot(pooled * coeff, cw_ref[...], preferred_element_type=F32)
    h_ref[...] = jnp.maximum(nl + cb_ref[...] + x_ref[...], 0.0)


def _tc_update(pooled, deg, cw, cb, x):
    return pl.pallas_call(
        _update_body,
        grid=(NP // BR,),
        in_specs=[
            pl.BlockSpec((NC, BR, EMBED), lambda i: (0, i, 0)),
            pl.BlockSpec((NC, BR, 8), lambda i: (0, i, 0)),
            pl.BlockSpec((EMBED, EMBED), lambda i: (0, 0)),
            pl.BlockSpec((1, EMBED), lambda i: (0, 0)),
            pl.BlockSpec((BR, EMBED), lambda i: (i, 0)),
        ],
        out_specs=pl.BlockSpec((BR, EMBED), lambda i: (i, 0)),
        out_shape=jax.ShapeDtypeStruct((NP, EMBED), F32),
    )(pooled, deg, cw, cb, x)


def _reduce_body(tn_ref, h_ref, s_ref, t_ref):
    i = pl.program_id(0)

    @pl.when(i == 0)
    def _():
        s_ref[...] = jnp.zeros_like(s_ref)
        t_ref[...] = jnp.zeros_like(t_ref)

    h = h_ref[...]
    gid = i * BR + lax.broadcasted_iota(jnp.int32, (BR, 1), 0)
    valid = jnp.where(gid < N, 1.0, 0.0)
    s_ref[...] += jnp.sum(h * valid, axis=0, keepdims=True)
    tm = jnp.where(gid == tn_ref[0], 1.0, 0.0)
    t_ref[...] += jnp.sum(h * tm, axis=0, keepdims=True)


def _tc_reduce(h, tn):
    return pl.pallas_call(
        _reduce_body,
        grid=(NP // BR,),
        in_specs=[
            pl.BlockSpec(memory_space=pltpu.SMEM),
            pl.BlockSpec((BR, EMBED), lambda i: (i, 0)),
        ],
        out_specs=[pl.BlockSpec((1, EMBED), lambda i: (0, 0))] * 2,
        out_shape=[jax.ShapeDtypeStruct((1, EMBED), F32)] * 2,
    )(tn, h)


def _score_body(h_ref, wa_ref, wb_ref, lb_ref, ow_ref, ob_ref, s_ref, t_ref,
                q_ref):
    g = s_ref[...] * (1.0 / N)
    beff = jnp.dot(g, wb_ref[...], preferred_element_type=F32) + lb_ref[...]
    hh = jnp.maximum(
        jnp.dot(h_ref[...], wa_ref[...], preferred_element_type=F32) + beff,
        0.0)
    ro = jnp.dot(hh, ow_ref[...], preferred_element_type=F32) + ob_ref[...]
    t = t_ref[...]
    q_ref[...] = lax.dot_general(ro, t, (((1,), (1,)), ((), ())),
                                 preferred_element_type=F32)


def _tc_score(h, wa, wb, lb, ow, ob, s, t):
    return pl.pallas_call(
        _score_body,
        grid=(NP // BR,),
        in_specs=[
            pl.BlockSpec((BR, EMBED), lambda i: (i, 0)),
            pl.BlockSpec((EMBED, EMBED), lambda i: (0, 0)),
            pl.BlockSpec((EMBED, EMBED), lambda i: (0, 0)),
            pl.BlockSpec((1, EMBED), lambda i: (0, 0)),
            pl.BlockSpec((EMBED, EMBED), lambda i: (0, 0)),
            pl.BlockSpec((1, EMBED), lambda i: (0, 0)),
            pl.BlockSpec((1, EMBED), lambda i: (0, 0)),
            pl.BlockSpec((1, EMBED), lambda i: (0, 0)),
        ],
        out_specs=pl.BlockSpec((BR, 1), lambda i: (i, 0)),
        out_shape=jax.ShapeDtypeStruct((NP, 1), F32),
    )(h, wa, wb, lb, ow, ob, s, t)


def kernel(node_features, edge_index, w_n2l, bias_n2l, bias_picked, conv_W,
           conv_b, lin1_W, lin1_b, out_W, out_b, target_node, picked_node):
    nf = jnp.concatenate(
        [node_features, jnp.zeros((NP - N, D_FEAT), F32)], axis=0)
    src = jnp.concatenate(
        [edge_index[0], jnp.zeros((EP - E,), jnp.int32)])
    dst = jnp.concatenate(
        [edge_index[1], jnp.full((EP - E,), N, jnp.int32)])
    srcr = src.reshape(NW, CPW, CHUNK)
    dstr = dst.reshape(NW, CPW, CHUNK)
    z64 = jnp.zeros((RPS, EMBED), F32)
    z8 = jnp.zeros((RPS, 8), F32)
    ones8 = jnp.ones((CHUNK, 8), F32)
    b_n2l = bias_n2l.reshape(1, EMBED)
    cb = conv_b.reshape(1, EMBED)
    lb = lin1_b.reshape(1, EMBED)
    ob = out_b.reshape(1, EMBED)
    pk = jnp.asarray(picked_node, jnp.int32).reshape(1)
    tn = jnp.asarray(target_node, jnp.int32).reshape(1)

    x, h0 = _tc_embed(nf, w_n2l, b_n2l, bias_picked, pk)
    pooled1, deg = _sc_spmm_deg(h0, srcr, dstr, z64, z8, ones8)
    h1 = _tc_update(pooled1, deg, conv_W, cb, x)
    pooled2 = _sc_spmm(h1, srcr, dstr, z64, z8, ones8)
    if isinstance(pooled2, (list, tuple)):
        pooled2 = pooled2[0]
    h2 = _tc_update(pooled2, deg, conv_W, cb, x)
    s, t = _tc_reduce(h2, tn)
    q = _tc_score(h2, lin1_W[:EMBED], lin1_W[EMBED:], lb, out_W, ob, s, t)
    return q[:N]


# speedup vs baseline: 4.0915x; 4.0915x over previous
import jax
import jax.numpy as jnp
from jax import lax
from jax.experimental import pallas as pl
from jax.experimental.pallas import tpu as pltpu
from jax.experimental.pallas import tpu_sc as plsc

N = 10000
NP = 10240
E = 320000
D_FEAT = 128
EMBED = 64
NC = 2
NS = 16
NW = NC * NS
CHUNK = 128
CPW = 80
EP = NW * CPW * CHUNK
RPS = NP // NS
BR = 1280
F32 = jnp.float32


def _make_sc_spmm(with_deg: bool):
    mesh = plsc.VectorSubcoreMesh(core_axis_name="c", subcore_axis_name="s",
                                  num_cores=NC, num_subcores=NS)
    out_type = [jax.ShapeDtypeStruct((NC, NP, EMBED), F32)]
    scratch = [
        pltpu.VMEM_SHARED((NP, EMBED), F32),
        pltpu.VMEM((CHUNK,), jnp.int32),
        pltpu.VMEM((CHUNK,), jnp.int32),
        pltpu.VMEM((CHUNK, EMBED), F32),
        pltpu.SemaphoreType.DMA,
    ]
    if with_deg:
        out_type.append(jax.ShapeDtypeStruct((NC, NP, 8), F32))
        scratch += [
            pltpu.VMEM_SHARED((NP, 8), F32),
            pltpu.VMEM((CHUNK, 8), F32),
        ]

    def body(h_hbm, src_hbm, dst_hbm, z64_hbm, z8_hbm, ones_hbm, *rest):
        if with_deg:
            (pooled_out, deg_out, pooled_sh, sidx, didx, rows, sem,
             deg_sh, ones_v) = rest
        else:
            pooled_out, pooled_sh, sidx, didx, rows, sem = rest
        cid = lax.axis_index("c")
        sid = lax.axis_index("s")
        wid = sid * NC + cid
        r0 = sid * RPS

        pltpu.sync_copy(z64_hbm, pooled_sh.at[pl.ds(r0, RPS)])
        if with_deg:
            pltpu.sync_copy(z8_hbm, deg_sh.at[pl.ds(r0, RPS)])
            pltpu.sync_copy(ones_hbm, ones_v)
        plsc.subcore_barrier()

        def step(j, carry):
            pltpu.sync_copy(src_hbm.at[wid, j], sidx)
            pltpu.async_copy(h_hbm.at[sidx], rows, sem).wait()
            pltpu.sync_copy(dst_hbm.at[wid, j], didx)
            pltpu.sync_copy(rows, pooled_sh.at[didx], add=True)
            if with_deg:
                pltpu.sync_copy(ones_v, deg_sh.at[didx], add=True)
            return carry

        lax.fori_loop(0, CPW, step, 0)
        plsc.subcore_barrier()

        pltpu.sync_copy(pooled_sh.at[pl.ds(r0, RPS)],
                        pooled_out.at[cid, pl.ds(r0, RPS)])
        if with_deg:
            pltpu.sync_copy(deg_sh.at[pl.ds(r0, RPS)],
                            deg_out.at[cid, pl.ds(r0, RPS)])

    return pl.kernel(
        body, out_type=out_type, mesh=mesh, scratch_types=scratch,
        compiler_params=pltpu.CompilerParams(use_tc_tiling_on_sc=False))


_sc_cache = {}


def _get_sc_spmm(with_deg: bool):
    if with_deg not in _sc_cache:
        _sc_cache[with_deg] = _make_sc_spmm(with_deg)
    return _sc_cache[with_deg]


def _sc_spmm_deg(*args):
    return _get_sc_spmm(True)(*args)


def _sc_spmm(*args):
    return _get_sc_spmm(False)(*args)


def _embed_body(pk_ref, nf_ref, w_ref, b_ref, bp_ref, x_ref, h_ref):
    i = pl.program_id(0)
    x = jnp.dot(nf_ref[...], w_ref[...], preferred_element_type=F32) + b_ref[...]
    gid = i * BR + lax.broadcasted_iota(jnp.int32, (BR, 1), 0)
    x = x + jnp.where(gid == pk_ref[0], 1.0, 0.0) * bp_ref[...]
    x_ref[...] = x
    h_ref[...] = jnp.maximum(x, 0.0)


def _tc_embed(nf, w, b, bp, pk):
    return pl.pallas_call(
        _embed_body,
        grid=(NP // BR,),
        in_specs=[
            pl.BlockSpec(memory_space=pltpu.SMEM),
            pl.BlockSpec((BR, D_FEAT), lambda i: (i, 0)),
            pl.BlockSpec((D_FEAT, EMBED), lambda i: (0, 0)),
            pl.BlockSpec((1, EMBED), lambda i: (0, 0)),
            pl.BlockSpec((1, EMBED), lambda i: (0, 0)),
        ],
        out_specs=[pl.BlockSpec((BR, EMBED), lambda i: (i, 0))] * 2,
        out_shape=[jax.ShapeDtypeStruct((NP, EMBED), F32)] * 2,
    )(pk, nf, w, b, bp)


def _update_body(p_ref, d_ref, cw_ref, cb_ref, x_ref, h_ref):
    pooled = p_ref[0] + p_ref[1]
    deg = d_ref[0][:, 0:1] + d_ref[1][:, 0:1]
    coeff = 1.0 / jnp.maximum(deg, 1.0)
    nl = jnp.dot(pooled * coeff, cw_ref[...], preferred_element_type=F32)
    h_ref[...] = jnp.maximum(nl + cb_ref[...] + x_ref[...], 0.0)


def _tc_update(pooled, deg, cw, cb, x):
    return pl.pallas_call(
        _update_body,
        grid=(NP // BR,),
        in_specs=[
            pl.BlockSpec((NC, BR, EMBED), lambda i: (0, i, 0)),
            pl.BlockSpec((NC, BR, 8), lambda i: (0, i, 0)),
            pl.BlockSpec((EMBED, EMBED), lambda i: (0, 0)),
            pl.BlockSpec((1, EMBED), lambda i: (0, 0)),
            pl.BlockSpec((BR, EMBED), lambda i: (i, 0)),
        ],
        out_specs=pl.BlockSpec((BR, EMBED), lambda i: (i, 0)),
        out_shape=jax.ShapeDtypeStruct((NP, EMBED), F32),
    )(pooled, deg, cw, cb, x)


def _reduce_body(tn_ref, h_ref, s_ref, t_ref):
    i = pl.program_id(0)

    @pl.when(i == 0)
    def _():
        s_ref[...] = jnp.zeros_like(s_ref)
        t_ref[...] = jnp.zeros_like(t_ref)

    h = h_ref[...]
    gid = i * BR + lax.broadcasted_iota(jnp.int32, (BR, 1), 0)
    valid = jnp.where(gid < N, 1.0, 0.0)
    s_ref[...] += jnp.sum(h * valid, axis=0, keepdims=True)
    tm = jnp.where(gid == tn_ref[0], 1.0, 0.0)
    t_ref[...] += jnp.sum(h * tm, axis=0, keepdims=True)


def _tc_reduce(h, tn):
    return pl.pallas_call(
        _reduce_body,
        grid=(NP // BR,),
        in_specs=[
            pl.BlockSpec(memory_space=pltpu.SMEM),
            pl.BlockSpec((BR, EMBED), lambda i: (i, 0)),
        ],
        out_specs=[pl.BlockSpec((1, EMBED), lambda i: (0, 0))] * 2,
        out_shape=[jax.ShapeDtypeStruct((1, EMBED), F32)] * 2,
    )(tn, h)


def _score_body(h_ref, wa_ref, wb_ref, lb_ref, ow_ref, ob_ref, s_ref, t_ref,
                q_ref):
    g = s_ref[...] * (1.0 / N)
    beff = jnp.dot(g, wb_ref[...], preferred_element_type=F32) + lb_ref[...]
    hh = jnp.maximum(
        jnp.dot(h_ref[...], wa_ref[...], preferred_element_type=F32) + beff,
        0.0)
    ro = jnp.dot(hh, ow_ref[...], preferred_element_type=F32) + ob_ref[...]
    t = t_ref[...]
    q_ref[...] = lax.dot_general(ro, t, (((1,), (1,)), ((), ())),
                                 preferred_element_type=F32)


def _tc_score(h, wa, wb, lb, ow, ob, s, t):
    return pl.pallas_call(
        _score_body,
        grid=(NP // BR,),
        in_specs=[
            pl.BlockSpec((BR, EMBED), lambda i: (i, 0)),
            pl.BlockSpec((EMBED, EMBED), lambda i: (0, 0)),
            pl.BlockSpec((EMBED, EMBED), lambda i: (0, 0)),
            pl.BlockSpec((1, EMBED), lambda i: (0, 0)),
            pl.BlockSpec((EMBED, EMBED), lambda i: (0, 0)),
            pl.BlockSpec((1, EMBED), lambda i: (0, 0)),
            pl.BlockSpec((1, EMBED), lambda i: (0, 0)),
            pl.BlockSpec((1, EMBED), lambda i: (0, 0)),
        ],
        out_specs=pl.BlockSpec((BR, 1), lambda i: (i, 0)),
        out_shape=jax.ShapeDtypeStruct((NP, 1), F32),
    )(h, wa, wb, lb, ow, ob, s, t)


def kernel(node_features, edge_index, w_n2l, bias_n2l, bias_picked, conv_W,
           conv_b, lin1_W, lin1_b, out_W, out_b, target_node, picked_node):
    nf = jnp.concatenate(
        [node_features, jnp.zeros((NP - N, D_FEAT), F32)], axis=0)
    src = jnp.concatenate(
        [edge_index[0], jnp.zeros((EP - E,), jnp.int32)])
    dst = jnp.concatenate(
        [edge_index[1], jnp.full((EP - E,), N, jnp.int32)])
    srcr = src.reshape(NW, CPW, CHUNK)
    dstr = dst.reshape(NW, CPW, CHUNK)
    z64 = jnp.zeros((RPS, EMBED), F32)
    z8 = jnp.zeros((RPS, 8), F32)
    ones8 = jnp.ones((CHUNK, 8), F32)
    b_n2l = bias_n2l.reshape(1, EMBED)
    cb = conv_b.reshape(1, EMBED)
    lb = lin1_b.reshape(1, EMBED)
    ob = out_b.reshape(1, EMBED)
    pk = jnp.asarray(picked_node, jnp.int32).reshape(1)
    tn = jnp.asarray(target_node, jnp.int32).reshape(1)

    x, h0 = _tc_embed(nf, w_n2l, b_n2l, bias_picked, pk)
    pooled1, deg = _sc_spmm_deg(h0, srcr, dstr, z64, z8, ones8)
    h1 = _tc_update(pooled1, deg, conv_W, cb, x)
    pooled2 = _sc_spmm(h1, srcr, dstr, z64, z8, ones8)
    if isinstance(pooled2, (list, tuple)):
        pooled2 = pooled2[0]
    h2 = _tc_update(pooled2, deg, conv_W, cb, x)
    s, t = _tc_reduce(h2, tn)
    q = _tc_score(h2, lin1_W[:EMBED], lin1_W[EMBED:], lb, out_W, ob, s, t)
    return q[:N]


# speedup vs baseline: 5.4128x; 1.3229x over previous
import jax
import jax.numpy as jnp
from jax import lax
from jax.experimental import pallas as pl
from jax.experimental.pallas import tpu as pltpu
from jax.experimental.pallas import tpu_sc as plsc

N = 10000
NP = 10240
E = 320000
D_FEAT = 128
EMBED = 64
NC = 2
NS = 16
NW = NC * NS
CHUNK = 128
CPW = 80
EP = NW * CPW * CHUNK
RPS = NP // NS
NB = 5
PF = 3
BR = 2000
F32 = jnp.float32


def _make_sc_spmm(with_deg: bool):
    mesh = plsc.VectorSubcoreMesh(core_axis_name="c", subcore_axis_name="s",
                                  num_cores=NC, num_subcores=NS)
    out_type = [jax.ShapeDtypeStruct((NC, NP, EMBED), F32)]
    scratch = [
        pltpu.VMEM_SHARED((NP, EMBED), F32),
        pltpu.VMEM((CPW, CHUNK), jnp.int32),
        pltpu.VMEM((CPW, CHUNK), jnp.int32),
        pltpu.VMEM((NB, CHUNK, EMBED), F32),
    ] + [pltpu.SemaphoreType.DMA] * NB
    if with_deg:
        out_type.append(jax.ShapeDtypeStruct((NC, NP, 8), F32))
        scratch += [
            pltpu.SemaphoreType.DMA,
            pltpu.VMEM_SHARED((NP, 8), F32),
            pltpu.VMEM((CHUNK, 8), F32),
        ]

    def body(h_hbm, src_hbm, dst_hbm, z64_hbm, z8_hbm, ones_hbm, *rest):
        if with_deg:
            (pooled_out, deg_out, pooled_sh, sidx, didx, rows,
             *sems, dsem, deg_sh, ones_v) = rest
        else:
            (pooled_out, pooled_sh, sidx, didx, rows, *sems) = rest
        cid = lax.axis_index("c")
        sid = lax.axis_index("s")
        wid = sid * NC + cid
        r0 = sid * RPS

        def start_g(j, b):
            pltpu.async_copy(h_hbm.at[sidx.at[j]], rows.at[b], sems[b])

        def wait_g(b):
            pltpu.make_async_copy(h_hbm.at[sidx.at[0]], rows.at[b],
                                  sems[b]).wait()

        def start_s(j, b):
            pltpu.async_copy(rows.at[b], pooled_sh.at[didx.at[j]], sems[b],
                             add=True)
            if with_deg:
                pltpu.async_copy(ones_v, deg_sh.at[didx.at[j]], dsem,
                                 add=True)

        def wait_s(b):
            pltpu.make_async_copy(rows.at[b], pooled_sh.at[didx.at[0]],
                                  sems[b]).wait()

        pltpu.sync_copy(z64_hbm, pooled_sh.at[pl.ds(r0, RPS)])
        pltpu.sync_copy(src_hbm.at[wid], sidx)
        pltpu.sync_copy(dst_hbm.at[wid], didx)
        if with_deg:
            pltpu.sync_copy(z8_hbm, deg_sh.at[pl.ds(r0, RPS)])
            pltpu.sync_copy(ones_hbm, ones_v)
        plsc.subcore_barrier()

        for b in range(PF):
            start_g(b, b)

        def outer(jo, carry):
            for b in range(NB):
                j = jo * NB + b
                bp = (b + PF) % NB

                @pl.when(j + PF < CPW)
                def _():
                    @pl.when(j >= NB - PF)
                    def _():
                        wait_s(bp)
                    start_g(j + PF, bp)

                wait_g(b)
                start_s(j, b)
            return carry

        lax.fori_loop(0, CPW // NB, outer, 0)

        for b in range(NB):
            wait_s(b)
        if with_deg:
            def drain_deg(j, carry):
                pltpu.make_async_copy(ones_v, deg_sh.at[didx.at[0]],
                                      dsem).wait()
                return carry
            lax.fori_loop(0, CPW, drain_deg, 0)
        plsc.subcore_barrier()

        pltpu.sync_copy(pooled_sh.at[pl.ds(r0, RPS)],
                        pooled_out.at[cid, pl.ds(r0, RPS)])
        if with_deg:
            pltpu.sync_copy(deg_sh.at[pl.ds(r0, RPS)],
                            deg_out.at[cid, pl.ds(r0, RPS)])

    return pl.kernel(
        body, out_type=out_type, mesh=mesh, scratch_types=scratch,
        compiler_params=pltpu.CompilerParams(use_tc_tiling_on_sc=False))


_sc_cache = {}


def _get_sc_spmm(with_deg: bool):
    if with_deg not in _sc_cache:
        _sc_cache[with_deg] = _make_sc_spmm(with_deg)
    return _sc_cache[with_deg]


def _sc_spmm_deg(*args):
    return _get_sc_spmm(True)(*args)


def _sc_spmm(*args):
    return _get_sc_spmm(False)(*args)


def _embed_body(pk_ref, nf_ref, w_ref, b_ref, bp_ref, x_ref, h_ref):
    i = pl.program_id(0)
    x = jnp.dot(nf_ref[...], w_ref[...], preferred_element_type=F32) + b_ref[...]
    gid = i * BR + lax.broadcasted_iota(jnp.int32, (BR, 1), 0)
    x = x + jnp.where(gid == pk_ref[0], 1.0, 0.0) * bp_ref[...]
    x_ref[...] = x
    h_ref[...] = jnp.maximum(x, 0.0)


def _tc_embed(nf, w, b, bp, pk):
    return pl.pallas_call(
        _embed_body,
        grid=(N // BR,),
        in_specs=[
            pl.BlockSpec(memory_space=pltpu.SMEM),
            pl.BlockSpec((BR, D_FEAT), lambda i: (i, 0)),
            pl.BlockSpec((D_FEAT, EMBED), lambda i: (0, 0)),
            pl.BlockSpec((1, EMBED), lambda i: (0, 0)),
            pl.BlockSpec((1, EMBED), lambda i: (0, 0)),
        ],
        out_specs=[pl.BlockSpec((BR, EMBED), lambda i: (i, 0))] * 2,
        out_shape=[jax.ShapeDtypeStruct((N, EMBED), F32)] * 2,
    )(pk, nf, w, b, bp)


def _update_body(p_ref, d_ref, cw_ref, cb_ref, x_ref, h_ref):
    pooled = p_ref[0] + p_ref[1]
    deg = d_ref[0][:, 0:1] + d_ref[1][:, 0:1]
    coeff = 1.0 / jnp.maximum(deg, 1.0)
    nl = jnp.dot(pooled * coeff, cw_ref[...], preferred_element_type=F32)
    h_ref[...] = jnp.maximum(nl + cb_ref[...] + x_ref[...], 0.0)


def _tc_update(pooled, deg, cw, cb, x):
    return pl.pallas_call(
        _update_body,
        grid=(N // BR,),
        in_specs=[
            pl.BlockSpec((NC, BR, EMBED), lambda i: (0, i, 0)),
            pl.BlockSpec((NC, BR, 8), lambda i: (0, i, 0)),
            pl.BlockSpec((EMBED, EMBED), lambda i: (0, 0)),
            pl.BlockSpec((1, EMBED), lambda i: (0, 0)),
            pl.BlockSpec((BR, EMBED), lambda i: (i, 0)),
        ],
        out_specs=pl.BlockSpec((BR, EMBED), lambda i: (i, 0)),
        out_shape=jax.ShapeDtypeStruct((N, EMBED), F32),
    )(pooled, deg, cw, cb, x)


def _reduce_body(tn_ref, h_ref, s_ref, t_ref):
    i = pl.program_id(0)

    @pl.when(i == 0)
    def _():
        s_ref[...] = jnp.zeros_like(s_ref)
        t_ref[...] = jnp.zeros_like(t_ref)

    h = h_ref[...]
    s_ref[...] += jnp.sum(h, axis=0, keepdims=True)
    gid = i * BR + lax.broadcasted_iota(jnp.int32, (BR, 1), 0)
    tm = jnp.where(gid == tn_ref[0], 1.0, 0.0)
    t_ref[...] += jnp.sum(h * tm, axis=0, keepdims=True)


def _tc_reduce(h, tn):
    return pl.pallas_call(
        _reduce_body,
        grid=(N // BR,),
        in_specs=[
            pl.BlockSpec(memory_space=pltpu.SMEM),
            pl.BlockSpec((BR, EMBED), lambda i: (i, 0)),
        ],
        out_specs=[pl.BlockSpec((1, EMBED), lambda i: (0, 0))] * 2,
        out_shape=[jax.ShapeDtypeStruct((1, EMBED), F32)] * 2,
    )(tn, h)


def _score_body(h_ref, wa_ref, wb_ref, lb_ref, ow_ref, ob_ref, s_ref, t_ref,
                q_ref):
    g = s_ref[...] * (1.0 / N)
    beff = jnp.dot(g, wb_ref[...], preferred_element_type=F32) + lb_ref[...]
    hh = jnp.maximum(
        jnp.dot(h_ref[...], wa_ref[...], preferred_element_type=F32) + beff,
        0.0)
    ro = jnp.dot(hh, ow_ref[...], preferred_element_type=F32) + ob_ref[...]
    t = t_ref[...]
    q_ref[...] = lax.dot_general(ro, t, (((1,), (1,)), ((), ())),
                                 preferred_element_type=F32)


def _tc_score(h, wa, wb, lb, ow, ob, s, t):
    return pl.pallas_call(
        _score_body,
        grid=(N // BR,),
        in_specs=[
            pl.BlockSpec((BR, EMBED), lambda i: (i, 0)),
            pl.BlockSpec((EMBED, EMBED), lambda i: (0, 0)),
            pl.BlockSpec((EMBED, EMBED), lambda i: (0, 0)),
            pl.BlockSpec((1, EMBED), lambda i: (0, 0)),
            pl.BlockSpec((EMBED, EMBED), lambda i: (0, 0)),
            pl.BlockSpec((1, EMBED), lambda i: (0, 0)),
            pl.BlockSpec((1, EMBED), lambda i: (0, 0)),
            pl.BlockSpec((1, EMBED), lambda i: (0, 0)),
        ],
        out_specs=pl.BlockSpec((BR, 1), lambda i: (i, 0)),
        out_shape=jax.ShapeDtypeStruct((N, 1), F32),
    )(h, wa, wb, lb, ow, ob, s, t)


def kernel(node_features, edge_index, w_n2l, bias_n2l, bias_picked, conv_W,
           conv_b, lin1_W, lin1_b, out_W, out_b, target_node, picked_node):
    src = jnp.concatenate(
        [edge_index[0], jnp.zeros((EP - E,), jnp.int32)])
    dst = jnp.concatenate(
        [edge_index[1], jnp.full((EP - E,), N, jnp.int32)])
    srcr = src.reshape(NW, CPW, CHUNK)
    dstr = dst.reshape(NW, CPW, CHUNK)
    z64 = jnp.zeros((RPS, EMBED), F32)
    z8 = jnp.zeros((RPS, 8), F32)
    ones8 = jnp.ones((CHUNK, 8), F32)
    b_n2l = bias_n2l.reshape(1, EMBED)
    cb = conv_b.reshape(1, EMBED)
    lb = lin1_b.reshape(1, EMBED)
    ob = out_b.reshape(1, EMBED)
    pk = jnp.asarray(picked_node, jnp.int32).reshape(1)
    tn = jnp.asarray(target_node, jnp.int32).reshape(1)

    x, h0 = _tc_embed(node_features, w_n2l, b_n2l, bias_picked, pk)
    pooled1, deg = _sc_spmm_deg(h0, srcr, dstr, z64, z8, ones8)
    h1 = _tc_update(pooled1, deg, conv_W, cb, x)
    pooled2 = _sc_spmm(h1, srcr, dstr, z64, z8, ones8)
    if isinstance(pooled2, (list, tuple)):
        pooled2 = pooled2[0]
    h2 = _tc_update(pooled2, deg, conv_W, cb, x)
    s, t = _tc_reduce(h2, tn)
    q = _tc_score(h2, lin1_W[:EMBED], lin1_W[EMBED:], lb, out_W, ob, s, t)
    return q


# speedup vs baseline: 6.3637x; 1.1757x over previous
import jax
import jax.numpy as jnp
from jax import lax
from jax.experimental import pallas as pl
from jax.experimental.pallas import tpu as pltpu
from jax.experimental.pallas import tpu_sc as plsc

N = 10000
NP = 10240
E = 320000
D_FEAT = 128
EMBED = 64
NC = 2
NS = 16
NW = NC * NS
CHUNK = 128
CPW0 = 128
CPW1 = 32
CPP = CPW0 + CPW1
EP = NS * CPP * CHUNK
RPS = NP // NS
NB = 8
PF = 4
PFI = 6
BR = 2000
F32 = jnp.float32


def _make_sc_spmm(with_deg: bool):
    mesh = plsc.VectorSubcoreMesh(core_axis_name="c", subcore_axis_name="s",
                                  num_cores=NC, num_subcores=NS)
    out_type = [jax.ShapeDtypeStruct((NC, NP, EMBED), F32)]
    scratch = [
        pltpu.VMEM_SHARED((NP, EMBED), F32),
        pltpu.VMEM((NB, CHUNK), jnp.int32),
        pltpu.VMEM((NB, CHUNK), jnp.int32),
        pltpu.VMEM((NB, CHUNK, EMBED), F32),
    ] + [pltpu.SemaphoreType.DMA] * (2 * NB)
    if with_deg:
        out_type.append(jax.ShapeDtypeStruct((NC, NP, 8), F32))
        scratch += [
            pltpu.VMEM_SHARED((NP, 8), F32),
            pltpu.VMEM((CHUNK, 8), F32),
        ]

    def body(h_hbm, src_hbm, dst_hbm, z64_hbm, z8_hbm, ones_hbm, *rest):
        if with_deg:
            (pooled_out, deg_out, pooled_sh, sidxb, didxb, rows,
             *allsems, deg_sh, ones_v) = rest
        else:
            (pooled_out, pooled_sh, sidxb, didxb, rows, *allsems) = rest
        sems, isems = allsems[:NB], allsems[NB:]
        cid = lax.axis_index("c")
        sid = lax.axis_index("s")
        r0 = sid * RPS
        base = jnp.where(cid == 0, 0, CPW0)
        cpw = jnp.where(cid == 0, CPW0, CPW1)

        def start_i(j, b):
            pltpu.async_copy(src_hbm.at[sid, base + j], sidxb.at[b], isems[b])
            pltpu.async_copy(dst_hbm.at[sid, base + j], didxb.at[b], isems[b])

        def wait_i(b):
            pltpu.make_async_copy(src_hbm.at[0, 0], sidxb.at[b],
                                  isems[b]).wait()
            pltpu.make_async_copy(dst_hbm.at[0, 0], didxb.at[b],
                                  isems[b]).wait()

        def start_g(b):
            pltpu.async_copy(h_hbm.at[sidxb.at[b]], rows.at[b], sems[b])

        def wait_g(b):
            pltpu.make_async_copy(h_hbm.at[sidxb.at[0]], rows.at[b],
                                  sems[b]).wait()

        def start_s(b):
            pltpu.async_copy(rows.at[b], pooled_sh.at[didxb.at[b]], sems[b],
                             add=True)
            if with_deg:
                pltpu.async_copy(ones_v, deg_sh.at[didxb.at[b]], sems[b],
                                 add=True)

        def wait_s(b):
            pltpu.make_async_copy(rows.at[b], pooled_sh.at[didxb.at[0]],
                                  sems[b]).wait()
            if with_deg:
                pltpu.make_async_copy(ones_v, deg_sh.at[didxb.at[0]],
                                      sems[b]).wait()

        pltpu.sync_copy(z64_hbm, pooled_sh.at[pl.ds(r0, RPS)])
        if with_deg:
            pltpu.sync_copy(z8_hbm, deg_sh.at[pl.ds(r0, RPS)])
            pltpu.sync_copy(ones_hbm, ones_v)
        plsc.subcore_barrier()

        for c in range(PFI):
            start_i(c, c % NB)
        for c in range(PF):
            wait_i(c)
            start_g(c)

        def outer(jo, carry):
            for b in range(NB):
                j = jo * NB + b
                bg = (b + PF) % NB
                bi = (b + PFI) % NB

                @pl.when(j + PFI < cpw)
                def _():
                    @pl.when(j + PFI >= NB)
                    def _():
                        wait_s(bi)
                    start_i(j + PFI, bi)

                @pl.when(j + PF < cpw)
                def _():
                    wait_i(bg)
                    start_g(bg)

                wait_g(b)
                start_s(b)
            return carry

        lax.fori_loop(0, cpw // NB, outer, 0)

        for b in range(NB):
            wait_s(b)
        plsc.subcore_barrier()

        pltpu.sync_copy(pooled_sh.at[pl.ds(r0, RPS)],
                        pooled_out.at[cid, pl.ds(r0, RPS)])
        if with_deg:
            pltpu.sync_copy(deg_sh.at[pl.ds(r0, RPS)],
                            deg_out.at[cid, pl.ds(r0, RPS)])

    return pl.kernel(
        body, out_type=out_type, mesh=mesh, scratch_types=scratch,
        compiler_params=pltpu.CompilerParams(use_tc_tiling_on_sc=False))


_sc_cache = {}


def _get_sc_spmm(with_deg: bool):
    if with_deg not in _sc_cache:
        _sc_cache[with_deg] = _make_sc_spmm(with_deg)
    return _sc_cache[with_deg]


def _sc_spmm_deg(*args):
    return _get_sc_spmm(True)(*args)


def _sc_spmm(*args):
    return _get_sc_spmm(False)(*args)


def _embed_body(pk_ref, nf_ref, w_ref, b_ref, bp_ref, x_ref, h_ref):
    i = pl.program_id(0)
    x = jnp.dot(nf_ref[...], w_ref[...], preferred_element_type=F32) + b_ref[...]
    gid = i * BR + lax.broadcasted_iota(jnp.int32, (BR, 1), 0)
    x = x + jnp.where(gid == pk_ref[0], 1.0, 0.0) * bp_ref[...]
    x_ref[...] = x
    h_ref[...] = jnp.maximum(x, 0.0)


def _tc_embed(nf, w, b, bp, pk):
    return pl.pallas_call(
        _embed_body,
        grid=(N // BR,),
        in_specs=[
            pl.BlockSpec(memory_space=pltpu.SMEM),
            pl.BlockSpec((BR, D_FEAT), lambda i: (i, 0)),
            pl.BlockSpec((D_FEAT, EMBED), lambda i: (0, 0)),
            pl.BlockSpec((1, EMBED), lambda i: (0, 0)),
            pl.BlockSpec((1, EMBED), lambda i: (0, 0)),
        ],
        out_specs=[pl.BlockSpec((BR, EMBED), lambda i: (i, 0))] * 2,
        out_shape=[jax.ShapeDtypeStruct((N, EMBED), F32)] * 2,
    )(pk, nf, w, b, bp)


def _update_body(p_ref, d_ref, cw_ref, cb_ref, x_ref, h_ref):
    pooled = p_ref[0] + p_ref[1]
    deg = d_ref[0][:, 0:1] + d_ref[1][:, 0:1]
    coeff = 1.0 / jnp.maximum(deg, 1.0)
    nl = jnp.dot(pooled * coeff, cw_ref[...], preferred_element_type=F32)
    h_ref[...] = jnp.maximum(nl + cb_ref[...] + x_ref[...], 0.0)


def _tc_update(pooled, deg, cw, cb, x):
    return pl.pallas_call(
        _update_body,
        grid=(N // BR,),
        in_specs=[
            pl.BlockSpec((NC, BR, EMBED), lambda i: (0, i, 0)),
            pl.BlockSpec((NC, BR, 8), lambda i: (0, i, 0)),
            pl.BlockSpec((EMBED, EMBED), lambda i: (0, 0)),
            pl.BlockSpec((1, EMBED), lambda i: (0, 0)),
            pl.BlockSpec((BR, EMBED), lambda i: (i, 0)),
        ],
        out_specs=pl.BlockSpec((BR, EMBED), lambda i: (i, 0)),
        out_shape=jax.ShapeDtypeStruct((N, EMBED), F32),
    )(pooled, deg, cw, cb, x)


def _reduce_body(tn_ref, h_ref, s_ref, t_ref):
    i = pl.program_id(0)

    @pl.when(i == 0)
    def _():
        s_ref[...] = jnp.zeros_like(s_ref)
        t_ref[...] = jnp.zeros_like(t_ref)

    h = h_ref[...]
    s_ref[...] += jnp.sum(h, axis=0, keepdims=True)
    gid = i * BR + lax.broadcasted_iota(jnp.int32, (BR, 1), 0)
    tm = jnp.where(gid == tn_ref[0], 1.0, 0.0)
    t_ref[...] += jnp.sum(h * tm, axis=0, keepdims=True)


def _tc_reduce(h, tn):
    return pl.pallas_call(
        _reduce_body,
        grid=(N // BR,),
        in_specs=[
            pl.BlockSpec(memory_space=pltpu.SMEM),
            pl.BlockSpec((BR, EMBED), lambda i: (i, 0)),
        ],
        out_specs=[pl.BlockSpec((1, EMBED), lambda i: (0, 0))] * 2,
        out_shape=[jax.ShapeDtypeStruct((1, EMBED), F32)] * 2,
    )(tn, h)


def _score_body(h_ref, wa_ref, wb_ref, lb_ref, ow_ref, ob_ref, s_ref, t_ref,
                q_ref):
    g = s_ref[...] * (1.0 / N)
    beff = jnp.dot(g, wb_ref[...], preferred_element_type=F32) + lb_ref[...]
    hh = jnp.maximum(
        jnp.dot(h_ref[...], wa_ref[...], preferred_element_type=F32) + beff,
        0.0)
    ro = jnp.dot(hh, ow_ref[...], preferred_element_type=F32) + ob_ref[...]
    t = t_ref[...]
    q_ref[...] = lax.dot_general(ro, t, (((1,), (1,)), ((), ())),
                                 preferred_element_type=F32)


def _tc_score(h, wa, wb, lb, ow, ob, s, t):
    return pl.pallas_call(
        _score_body,
        grid=(N // BR,),
        in_specs=[
            pl.BlockSpec((BR, EMBED), lambda i: (i, 0)),
            pl.BlockSpec((EMBED, EMBED), lambda i: (0, 0)),
            pl.BlockSpec((EMBED, EMBED), lambda i: (0, 0)),
            pl.BlockSpec((1, EMBED), lambda i: (0, 0)),
            pl.BlockSpec((EMBED, EMBED), lambda i: (0, 0)),
            pl.BlockSpec((1, EMBED), lambda i: (0, 0)),
            pl.BlockSpec((1, EMBED), lambda i: (0, 0)),
            pl.BlockSpec((1, EMBED), lambda i: (0, 0)),
        ],
        out_specs=pl.BlockSpec((BR, 1), lambda i: (i, 0)),
        out_shape=jax.ShapeDtypeStruct((N, 1), F32),
    )(h, wa, wb, lb, ow, ob, s, t)


def kernel(node_features, edge_index, w_n2l, bias_n2l, bias_picked, conv_W,
           conv_b, lin1_W, lin1_b, out_W, out_b, target_node, picked_node):
    src = jnp.concatenate(
        [edge_index[0], jnp.zeros((EP - E,), jnp.int32)])
    dst = jnp.concatenate(
        [edge_index[1], jnp.full((EP - E,), N, jnp.int32)])
    srcr = src.reshape(NS, CPP, CHUNK)
    dstr = dst.reshape(NS, CPP, CHUNK)
    z64 = jnp.zeros((RPS, EMBED), F32)
    z8 = jnp.zeros((RPS, 8), F32)
    ones8 = jnp.ones((CHUNK, 8), F32)
    b_n2l = bias_n2l.reshape(1, EMBED)
    cb = conv_b.reshape(1, EMBED)
    lb = lin1_b.reshape(1, EMBED)
    ob = out_b.reshape(1, EMBED)
    pk = jnp.asarray(picked_node, jnp.int32).reshape(1)
    tn = jnp.asarray(target_node, jnp.int32).reshape(1)

    x, h0 = _tc_embed(node_features, w_n2l, b_n2l, bias_picked, pk)
    pooled1, deg = _sc_spmm_deg(h0, srcr, dstr, z64, z8, ones8)
    h1 = _tc_update(pooled1, deg, conv_W, cb, x)
    pooled2 = _sc_spmm(h1, srcr, dstr, z64, z8, ones8)
    if isinstance(pooled2, (list, tuple)):
        pooled2 = pooled2[0]
    h2 = _tc_update(pooled2, deg, conv_W, cb, x)
    s, t = _tc_reduce(h2, tn)
    q = _tc_score(h2, lin1_W[:EMBED], lin1_W[EMBED:], lb, out_W, ob, s, t)
    return q


# speedup vs baseline: 15.0816x; 2.3699x over previous
import jax
import jax.numpy as jnp
from jax import lax
from jax.experimental import pallas as pl
from jax.experimental.pallas import tpu as pltpu
from jax.experimental.pallas import tpu_sc as plsc

N = 10000
NP = 10240
E = 320000
D_FEAT = 128
EMBED = 64
NC = 2
NS = 16
NW = NC * NS
CHUNK = 128
CPW0 = 80
CPP = NC * CPW0
NCHR = E // CHUNK
NPAD = NW * CPW0 - NCHR
RPS = NP // NS
NB = 5
PF = 3
BR = 2000
F32 = jnp.float32


def _make_sc_spmm(with_deg: bool):
    mesh = plsc.VectorSubcoreMesh(core_axis_name="c", subcore_axis_name="s",
                                  num_cores=NC, num_subcores=NS)
    out_type = [jax.ShapeDtypeStruct((NC, NP, EMBED), F32)]
    scratch = [
        pltpu.VMEM_SHARED((NP, EMBED), F32),
        pltpu.VMEM((CPW0, CHUNK), jnp.int32),
        pltpu.VMEM((CPW0, CHUNK), jnp.int32),
        pltpu.VMEM((NB, CHUNK, EMBED), F32),
    ] + [pltpu.SemaphoreType.DMA] * NB
    if with_deg:
        out_type.append(jax.ShapeDtypeStruct((NC, NP, 8), F32))
        scratch += [
            pltpu.VMEM_SHARED((NP, 8), F32),
            pltpu.VMEM((CHUNK, 8), F32),
        ]

    def body(h_hbm, ei_hbm, pads_hbm, z64_hbm, z8_hbm, ones_hbm, *rest):
        if with_deg:
            (pooled_out, deg_out, pooled_sh, sidx, didx, rows,
             *sems, deg_sh, ones_v) = rest
        else:
            (pooled_out, pooled_sh, sidx, didx, rows, *sems) = rest
        cid = lax.axis_index("c")
        sid = lax.axis_index("s")
        r0 = sid * RPS
        base = sid * CPP + cid * CPW0

        def start_g(j, b):
            pltpu.async_copy(h_hbm.at[sidx.at[j]], rows.at[b], sems[b])

        def wait_g(b):
            pltpu.make_async_copy(h_hbm.at[sidx.at[0]], rows.at[b],
                                  sems[b]).wait()

        def start_s(j, b):
            pltpu.async_copy(rows.at[b], pooled_sh.at[didx.at[j]],
                             sems[b], add=True)
            if with_deg:
                pltpu.async_copy(ones_v, deg_sh.at[didx.at[j]],
                                 sems[b], add=True)

        def wait_s(b):
            pltpu.make_async_copy(rows.at[b], pooled_sh.at[didx.at[0]],
                                  sems[b]).wait()
            if with_deg:
                pltpu.make_async_copy(ones_v, deg_sh.at[didx.at[0]],
                                      sems[b]).wait()

        pltpu.sync_copy(z64_hbm, pooled_sh.at[pl.ds(r0, RPS)])
        is_pad_worker = jnp.logical_and(sid == NS - 1, cid == 1)

        @pl.when(jnp.logical_not(is_pad_worker))
        def _():
            pltpu.sync_copy(ei_hbm.at[0, pl.ds(base, CPW0)], sidx)
            pltpu.sync_copy(ei_hbm.at[1, pl.ds(base, CPW0)], didx)

        @pl.when(is_pad_worker)
        def _():
            real = NCHR - (NW - 1) * CPW0
            pltpu.sync_copy(ei_hbm.at[0, pl.ds(NCHR - real, real)],
                            sidx.at[pl.ds(0, real)])
            pltpu.sync_copy(pads_hbm.at[0], sidx.at[pl.ds(real, NPAD)])
            pltpu.sync_copy(ei_hbm.at[1, pl.ds(NCHR - real, real)],
                            didx.at[pl.ds(0, real)])
            pltpu.sync_copy(pads_hbm.at[1], didx.at[pl.ds(real, NPAD)])

        if with_deg:
            pltpu.sync_copy(z8_hbm, deg_sh.at[pl.ds(r0, RPS)])
            pltpu.sync_copy(ones_hbm, ones_v)
        plsc.subcore_barrier()

        for b in range(PF):
            start_g(b, b)

        def outer(jo, carry):
            for b in range(NB):
                j = jo * NB + b
                bp = (b + PF) % NB

                @pl.when(j + PF < CPW0)
                def _():
                    @pl.when(j >= NB - PF)
                    def _():
                        wait_s(bp)
                    start_g(j + PF, bp)

                wait_g(b)
                start_s(j, b)
            return carry

        lax.fori_loop(0, CPW0 // NB, outer, 0)

        for b in range(NB):
            wait_s(b)
        plsc.subcore_barrier()

        pltpu.sync_copy(pooled_sh.at[pl.ds(r0, RPS)],
                        pooled_out.at[cid, pl.ds(r0, RPS)])
        if with_deg:
            pltpu.sync_copy(deg_sh.at[pl.ds(r0, RPS)],
                            deg_out.at[cid, pl.ds(r0, RPS)])

    return pl.kernel(
        body, out_type=out_type, mesh=mesh, scratch_types=scratch,
        compiler_params=pltpu.CompilerParams(use_tc_tiling_on_sc=False))


_sc_cache = {}


def _get_sc_spmm(with_deg: bool):
    if with_deg not in _sc_cache:
        _sc_cache[with_deg] = _make_sc_spmm(with_deg)
    return _sc_cache[with_deg]


def _sc_spmm_deg(*args):
    return _get_sc_spmm(True)(*args)


def _sc_spmm(*args):
    return _get_sc_spmm(False)(*args)


def _embed_body(pk_ref, nf_ref, w_ref, b_ref, bp_ref, x_ref, h_ref):
    i = pl.program_id(0)
    x = jnp.dot(nf_ref[...], w_ref[...], preferred_element_type=F32) + b_ref[...]
    gid = i * BR + lax.broadcasted_iota(jnp.int32, (BR, 1), 0)
    x = x + jnp.where(gid == pk_ref[0], 1.0, 0.0) * bp_ref[...]
    x_ref[...] = x
    h_ref[...] = jnp.maximum(x, 0.0)


def _tc_embed(nf, w, b, bp, pk):
    return pl.pallas_call(
        _embed_body,
        grid=(N // BR,),
        in_specs=[
            pl.BlockSpec(memory_space=pltpu.SMEM),
            pl.BlockSpec((BR, D_FEAT), lambda i: (i, 0)),
            pl.BlockSpec((D_FEAT, EMBED), lambda i: (0, 0)),
            pl.BlockSpec((1, EMBED), lambda i: (0, 0)),
            pl.BlockSpec((1, EMBED), lambda i: (0, 0)),
        ],
        out_specs=[pl.BlockSpec((BR, EMBED), lambda i: (i, 0))] * 2,
        out_shape=[jax.ShapeDtypeStruct((N, EMBED), F32)] * 2,
    )(pk, nf, w, b, bp)


def _update_body(p_ref, d_ref, cw_ref, cb_ref, x_ref, h_ref):
    pooled = p_ref[0] + p_ref[1]
    deg = d_ref[0][:, 0:1] + d_ref[1][:, 0:1]
    coeff = 1.0 / jnp.maximum(deg, 1.0)
    nl = jnp.dot(pooled * coeff, cw_ref[...], preferred_element_type=F32)
    h_ref[...] = jnp.maximum(nl + cb_ref[...] + x_ref[...], 0.0)


def _tc_update(pooled, deg, cw, cb, x):
    return pl.pallas_call(
        _update_body,
        grid=(N // BR,),
        in_specs=[
            pl.BlockSpec((NC, BR, EMBED), lambda i: (0, i, 0)),
            pl.BlockSpec((NC, BR, 8), lambda i: (0, i, 0)),
            pl.BlockSpec((EMBED, EMBED), lambda i: (0, 0)),
            pl.BlockSpec((1, EMBED), lambda i: (0, 0)),
            pl.BlockSpec((BR, EMBED), lambda i: (i, 0)),
        ],
        out_specs=pl.BlockSpec((BR, EMBED), lambda i: (i, 0)),
        out_shape=jax.ShapeDtypeStruct((N, EMBED), F32),
    )(pooled, deg, cw, cb, x)


def _update_reduce_body(tn_ref, p_ref, d_ref, cw_ref, cb_ref, x_ref,
                        h_ref, s_ref, t_ref):
    i = pl.program_id(0)
    pooled = p_ref[0] + p_ref[1]
    deg = d_ref[0][:, 0:1] + d_ref[1][:, 0:1]
    coeff = 1.0 / jnp.maximum(deg, 1.0)
    nl = jnp.dot(pooled * coeff, cw_ref[...], preferred_element_type=F32)
    h = jnp.maximum(nl + cb_ref[...] + x_ref[...], 0.0)
    h_ref[...] = h

    @pl.when(i == 0)
    def _():
        s_ref[...] = jnp.zeros_like(s_ref)
        t_ref[...] = jnp.zeros_like(t_ref)

    s_ref[...] += jnp.sum(h, axis=0, keepdims=True)
    gid = i * BR + lax.broadcasted_iota(jnp.int32, (BR, 1), 0)
    tm = jnp.where(gid == tn_ref[0], 1.0, 0.0)
    t_ref[...] += jnp.sum(h * tm, axis=0, keepdims=True)


def _tc_update_reduce(pooled, deg, cw, cb, x, tn):
    return pl.pallas_call(
        _update_reduce_body,
        grid=(N // BR,),
        in_specs=[
            pl.BlockSpec(memory_space=pltpu.SMEM),
            pl.BlockSpec((NC, BR, EMBED), lambda i: (0, i, 0)),
            pl.BlockSpec((NC, BR, 8), lambda i: (0, i, 0)),
            pl.BlockSpec((EMBED, EMBED), lambda i: (0, 0)),
            pl.BlockSpec((1, EMBED), lambda i: (0, 0)),
            pl.BlockSpec((BR, EMBED), lambda i: (i, 0)),
        ],
        out_specs=[
            pl.BlockSpec((BR, EMBED), lambda i: (i, 0)),
            pl.BlockSpec((1, EMBED), lambda i: (0, 0)),
            pl.BlockSpec((1, EMBED), lambda i: (0, 0)),
        ],
        out_shape=[
            jax.ShapeDtypeStruct((N, EMBED), F32),
            jax.ShapeDtypeStruct((1, EMBED), F32),
            jax.ShapeDtypeStruct((1, EMBED), F32),
        ],
    )(tn, pooled, deg, cw, cb, x)


def _score_body(h_ref, wa_ref, wb_ref, lb_ref, ow_ref, ob_ref, s_ref, t_ref,
                q_ref):
    g = s_ref[...] * (1.0 / N)
    beff = jnp.dot(g, wb_ref[...], preferred_element_type=F32) + lb_ref[...]
    hh = jnp.maximum(
        jnp.dot(h_ref[...], wa_ref[...], preferred_element_type=F32) + beff,
        0.0)
    ro = jnp.dot(hh, ow_ref[...], preferred_element_type=F32) + ob_ref[...]
    t = t_ref[...]
    q_ref[...] = lax.dot_general(ro, t, (((1,), (1,)), ((), ())),
                                 preferred_element_type=F32)


def _tc_score(h, wa, wb, lb, ow, ob, s, t):
    return pl.pallas_call(
        _score_body,
        grid=(N // BR,),
        in_specs=[
            pl.BlockSpec((BR, EMBED), lambda i: (i, 0)),
            pl.BlockSpec((EMBED, EMBED), lambda i: (0, 0)),
            pl.BlockSpec((EMBED, EMBED), lambda i: (0, 0)),
            pl.BlockSpec((1, EMBED), lambda i: (0, 0)),
            pl.BlockSpec((EMBED, EMBED), lambda i: (0, 0)),
            pl.BlockSpec((1, EMBED), lambda i: (0, 0)),
            pl.BlockSpec((1, EMBED), lambda i: (0, 0)),
            pl.BlockSpec((1, EMBED), lambda i: (0, 0)),
        ],
        out_specs=pl.BlockSpec((BR, 1), lambda i: (i, 0)),
        out_shape=jax.ShapeDtypeStruct((N, 1), F32),
    )(h, wa, wb, lb, ow, ob, s, t)


def kernel(node_features, edge_index, w_n2l, bias_n2l, bias_picked, conv_W,
           conv_b, lin1_W, lin1_b, out_W, out_b, target_node, picked_node):
    ei = edge_index.reshape(2, NCHR, CHUNK)
    pad_ids = (jnp.arange(NPAD * CHUNK, dtype=jnp.int32) % CHUNK)
    pads = jnp.stack([pad_ids, N + pad_ids]).reshape(2, NPAD, CHUNK)
    z64 = jnp.zeros((RPS, EMBED), F32)
    z8 = jnp.zeros((RPS, 8), F32)
    ones8 = jnp.ones((CHUNK, 8), F32)
    b_n2l = bias_n2l.reshape(1, EMBED)
    cb = conv_b.reshape(1, EMBED)
    lb = lin1_b.reshape(1, EMBED)
    ob = out_b.reshape(1, EMBED)
    pk = jnp.asarray(picked_node, jnp.int32).reshape(1)
    tn = jnp.asarray(target_node, jnp.int32).reshape(1)

    x, h0 = _tc_embed(node_features, w_n2l, b_n2l, bias_picked, pk)
    pooled1, deg = _sc_spmm_deg(h0, ei, pads, z64, z8, ones8)
    h1 = _tc_update(pooled1, deg, conv_W, cb, x)
    pooled2 = _sc_spmm(h1, ei, pads, z64, z8, ones8)
    if isinstance(pooled2, (list, tuple)):
        pooled2 = pooled2[0]
    h2, s, t = _tc_update_reduce(pooled2, deg, conv_W, cb, x, tn)
    q = _tc_score(h2, lin1_W[:EMBED], lin1_W[EMBED:], lb, out_W, ob, s, t)
    return q
